# baseline (speedup 1.0000x reference)
import functools

import jax
import jax.numpy as jnp
from jax import lax
from jax.experimental import pallas as pl
from jax.experimental.pallas import tpu as pltpu
from jax.experimental.pallas import tpu_sc as plsc

NC = 2
NS = 16
NW = NC * NS
GL = 128
K = 8
CH = K * GL


@functools.lru_cache(maxsize=None)
def _build(v, d, nch):
    mesh = plsc.VectorSubcoreMesh(core_axis_name="c", subcore_axis_name="s")

    @functools.partial(
        pl.kernel,
        mesh=mesh,
        out_type=jax.ShapeDtypeStruct((NW, nch, CH, d), jnp.float32),
        scratch_types=[
            pltpu.VMEM((nch * K, GL), jnp.int32),
            pltpu.VMEM((CH, d), jnp.float32),
            pltpu.SemaphoreType.DMA,
        ],
        compiler_params=pltpu.CompilerParams(use_tc_tiling_on_sc=False),
    )
    def k(table_hbm, tok_hbm, out_hbm, idx_v, rows_v, sem):
        wid = lax.axis_index("s") * NC + lax.axis_index("c")
        pltpu.sync_copy(tok_hbm.at[wid], idx_v)

        def chunk(c, carry):
            cps = [
                pltpu.async_copy(
                    table_hbm.at[idx_v.at[c * K + j]],
                    rows_v.at[pl.ds(j * GL, GL)],
                    sem,
                )
                for j in range(K)
            ]
            for cp in cps:
                cp.wait()
            pltpu.sync_copy(rows_v, out_hbm.at[wid, c])
            return carry

        lax.fori_loop(0, nch, chunk, 0)

    return k


def kernel(tokens, table):
    d = table.shape[1]
    flat = tokens.reshape(-1).astype(jnp.int32)
    b = flat.shape[0]
    blk = NW * CH
    pad = (-b) % blk
    if pad:
        flat = jnp.concatenate([flat, jnp.zeros((pad,), jnp.int32)])
    nch = flat.shape[0] // blk
    tok3 = flat.reshape(NW, nch * K, GL)
    out = _build(table.shape[0], d, nch)(table, tok3)
    out = out.reshape(-1, d)
    if pad:
        out = out[:b]
    return out
